# baseline (speedup 1.0000x reference)
import functools

import jax
import jax.numpy as jnp
from jax import lax
from jax.experimental import pallas as pl
from jax.experimental.pallas import tpu as pltpu
from jax.experimental.pallas import tpu_sc as plsc

B = 16384
D = 128
GAMMA = 12.0

NC = 2
NS = 16
L = 16
NW = NC * NS
BPW = B // NW
CHUNK = 128
NCHUNK = BPW // CHUNK
NG = CHUNK // L

_mesh = plsc.VectorSubcoreMesh(core_axis_name="c", subcore_axis_name="s")


@functools.partial(
    pl.kernel,
    out_type=jax.ShapeDtypeStruct((B,), jnp.float32),
    mesh=_mesh,
    compiler_params=pltpu.CompilerParams(needs_layout_passes=False),
    scratch_types=[
        pltpu.VMEM((CHUNK, 3), jnp.int32),
        pltpu.VMEM((CHUNK,), jnp.int32),
        pltpu.VMEM((CHUNK,), jnp.int32),
        pltpu.VMEM((CHUNK,), jnp.int32),
        pltpu.VMEM((CHUNK, D), jnp.float32),
        pltpu.VMEM((CHUNK, D), jnp.float32),
        pltpu.VMEM((CHUNK, D), jnp.float32),
        pltpu.VMEM((CHUNK,), jnp.float32),
        pltpu.SemaphoreType.DMA,
    ],
)
def _sc_score(samp_hbm, ent_hbm, rel_hbm, out_hbm,
              samp_v, idxh_v, idxr_v, idxt_v, hrows_v, rrows_v, trows_v,
              score_v, sem):
    wid = lax.axis_index("s") * NC + lax.axis_index("c")
    base = wid * BPW
    iota = lax.iota(jnp.int32, L)

    for c in range(NCHUNK):
        cb = base + c * CHUNK
        pltpu.sync_copy(samp_hbm.at[pl.ds(cb, CHUNK), :], samp_v)

        for g in range(NG):
            rows = g * L + iota
            idxh_v[pl.ds(g * L, L)] = plsc.load_gather(
                samp_v, [rows, jnp.full((L,), 0, jnp.int32)])
            idxr_v[pl.ds(g * L, L)] = plsc.load_gather(
                samp_v, [rows, jnp.full((L,), 1, jnp.int32)])
            idxt_v[pl.ds(g * L, L)] = plsc.load_gather(
                samp_v, [rows, jnp.full((L,), 2, jnp.int32)])

        ch = pltpu.async_copy(ent_hbm.at[idxh_v], hrows_v, sem)
        cr = pltpu.async_copy(rel_hbm.at[idxr_v], rrows_v, sem)
        ct = pltpu.async_copy(ent_hbm.at[idxt_v], trows_v, sem)
        ch.wait()
        cr.wait()
        ct.wait()

        for g in range(NG):
            rows = g * L + iota

            def dbody(dd, acc, rows=rows):
                col = jnp.full((L,), dd, jnp.int32)
                hv = plsc.load_gather(hrows_v, [rows, col])
                rv = plsc.load_gather(rrows_v, [rows, col])
                tv = plsc.load_gather(trows_v, [rows, col])
                return acc + jnp.abs(hv + rv - tv)

            acc = lax.fori_loop(0, D, dbody, jnp.zeros((L,), jnp.float32))
            score_v[pl.ds(g * L, L)] = GAMMA - acc

        pltpu.sync_copy(score_v, out_hbm.at[pl.ds(cb, CHUNK)])


def kernel(sample, entity_embedding, relation_embedding):
    scores = _sc_score(sample.astype(jnp.int32), entity_embedding,
                       relation_embedding)
    return scores[:, None]


# speedup vs baseline: 1.1224x; 1.1224x over previous
import functools

import jax
import jax.numpy as jnp
from jax import lax
from jax.experimental import pallas as pl
from jax.experimental.pallas import tpu as pltpu
from jax.experimental.pallas import tpu_sc as plsc

B = 16384
D = 128
GAMMA = 12.0

NC = 2
NS = 16
L = 16
NW = NC * NS
BPW = B // NW
CHUNK = 64
NCHUNK = BPW // CHUNK
NG = CHUNK // L
UNROLL = 16

_mesh = plsc.VectorSubcoreMesh(core_axis_name="c", subcore_axis_name="s")


@functools.partial(
    pl.kernel,
    out_type=jax.ShapeDtypeStruct((B,), jnp.float32),
    mesh=_mesh,
    compiler_params=pltpu.CompilerParams(needs_layout_passes=False),
    scratch_types=[
        pltpu.VMEM((2, CHUNK, 3), jnp.int32),
        pltpu.VMEM((2, CHUNK), jnp.int32),
        pltpu.VMEM((2, CHUNK), jnp.int32),
        pltpu.VMEM((2, CHUNK), jnp.int32),
        pltpu.VMEM((2, CHUNK, D), jnp.float32),
        pltpu.VMEM((2, CHUNK, D), jnp.float32),
        pltpu.VMEM((2, CHUNK, D), jnp.float32),
        pltpu.VMEM((CHUNK,), jnp.float32),
        pltpu.SemaphoreType.DMA,
        pltpu.SemaphoreType.DMA,
    ],
)
def _sc_score(samp_hbm, ent_hbm, rel_hbm, out_hbm,
              samp_v, idxh_v, idxr_v, idxt_v, hrows_v, rrows_v, trows_v,
              score_v, sem0, sem1):
    wid = lax.axis_index("s") * NC + lax.axis_index("c")
    base = wid * BPW
    iota = lax.iota(jnp.int32, L)
    sems = (sem0, sem1)

    def stage(c, buf):
        cb = base + c * CHUNK
        pltpu.sync_copy(samp_hbm.at[pl.ds(cb, CHUNK), :], samp_v.at[buf])
        for g in range(NG):
            rows = g * L + iota
            idxh_v[buf, pl.ds(g * L, L)] = plsc.load_gather(
                samp_v.at[buf], [rows, jnp.full((L,), 0, jnp.int32)])
            idxr_v[buf, pl.ds(g * L, L)] = plsc.load_gather(
                samp_v.at[buf], [rows, jnp.full((L,), 1, jnp.int32)])
            idxt_v[buf, pl.ds(g * L, L)] = plsc.load_gather(
                samp_v.at[buf], [rows, jnp.full((L,), 2, jnp.int32)])
        return (
            pltpu.async_copy(ent_hbm.at[idxh_v.at[buf]], hrows_v.at[buf],
                             sems[buf]),
            pltpu.async_copy(rel_hbm.at[idxr_v.at[buf]], rrows_v.at[buf],
                             sems[buf]),
            pltpu.async_copy(ent_hbm.at[idxt_v.at[buf]], trows_v.at[buf],
                             sems[buf]),
        )

    def score_chunk(c, buf):
        cb = base + c * CHUNK
        hb, rb, tb = hrows_v.at[buf], rrows_v.at[buf], trows_v.at[buf]
        for g in range(NG):
            rows = g * L + iota

            def dbody(dd, acc, rows=rows, hb=hb, rb=rb, tb=tb):
                d0 = dd * UNROLL
                for du in range(UNROLL):
                    col = jnp.full((L,), d0 + du, jnp.int32)
                    hv = plsc.load_gather(hb, [rows, col])
                    rv = plsc.load_gather(rb, [rows, col])
                    tv = plsc.load_gather(tb, [rows, col])
                    acc = acc + jnp.abs(hv + rv - tv)
                return acc

            acc = lax.fori_loop(0, D // UNROLL, dbody,
                                jnp.zeros((L,), jnp.float32))
            score_v[pl.ds(g * L, L)] = GAMMA - acc
        pltpu.sync_copy(score_v, out_hbm.at[pl.ds(cb, CHUNK)])

    copies = stage(0, 0)
    for c in range(NCHUNK):
        nxt = stage(c + 1, (c + 1) % 2) if c + 1 < NCHUNK else None
        for cp in copies:
            cp.wait()
        score_chunk(c, c % 2)
        copies = nxt


def kernel(sample, entity_embedding, relation_embedding):
    scores = _sc_score(sample.astype(jnp.int32), entity_embedding,
                       relation_embedding)
    return scores[:, None]


# speedup vs baseline: 2.8156x; 2.5086x over previous
import functools

import jax
import jax.numpy as jnp
from jax import lax
from jax.experimental import pallas as pl
from jax.experimental.pallas import tpu as pltpu
from jax.experimental.pallas import tpu_sc as plsc

B = 16384
D = 128
GAMMA = 12.0

NC = 2
NS = 16
L = 16
NW = NC * NS
BPW = B // NW
CHUNK = 64
NCHUNK = BPW // CHUNK
NG = CHUNK // L

_mesh = plsc.VectorSubcoreMesh(core_axis_name="c", subcore_axis_name="s")


@functools.partial(
    pl.kernel,
    out_type=jax.ShapeDtypeStruct((B,), jnp.float32),
    mesh=_mesh,
    compiler_params=pltpu.CompilerParams(needs_layout_passes=False),
    scratch_types=[
        pltpu.VMEM((2, CHUNK, 3), jnp.int32),
        pltpu.VMEM((2, CHUNK), jnp.int32),
        pltpu.VMEM((2, CHUNK), jnp.int32),
        pltpu.VMEM((2, CHUNK), jnp.int32),
        pltpu.VMEM((2, CHUNK, D), jnp.float32),
        pltpu.VMEM((2, CHUNK, D), jnp.float32),
        pltpu.VMEM((2, CHUNK, D), jnp.float32),
        pltpu.VMEM((CHUNK,), jnp.float32),
        pltpu.SemaphoreType.DMA,
        pltpu.SemaphoreType.DMA,
    ],
)
def _sc_score(samp_hbm, ent_hbm, rel_hbm, out_hbm,
              samp_v, idxh_v, idxr_v, idxt_v, hrows_v, rrows_v, trows_v,
              score_v, sem0, sem1):
    wid = lax.axis_index("s") * NC + lax.axis_index("c")
    base = wid * BPW
    iota = lax.iota(jnp.int32, L)
    sems = (sem0, sem1)

    def stage(c, buf):
        cb = base + c * CHUNK
        pltpu.sync_copy(samp_hbm.at[pl.ds(cb, CHUNK), :], samp_v.at[buf])
        for g in range(NG):
            rows = g * L + iota
            idxh_v[buf, pl.ds(g * L, L)] = plsc.load_gather(
                samp_v.at[buf], [rows, jnp.full((L,), 0, jnp.int32)])
            idxr_v[buf, pl.ds(g * L, L)] = plsc.load_gather(
                samp_v.at[buf], [rows, jnp.full((L,), 1, jnp.int32)])
            idxt_v[buf, pl.ds(g * L, L)] = plsc.load_gather(
                samp_v.at[buf], [rows, jnp.full((L,), 2, jnp.int32)])
        return (
            pltpu.async_copy(ent_hbm.at[idxh_v.at[buf]], hrows_v.at[buf],
                             sems[buf]),
            pltpu.async_copy(rel_hbm.at[idxr_v.at[buf]], rrows_v.at[buf],
                             sems[buf]),
            pltpu.async_copy(ent_hbm.at[idxt_v.at[buf]], trows_v.at[buf],
                             sems[buf]),
        )

    def score_chunk(c, buf):
        cb = base + c * CHUNK
        hb, rb, tb = hrows_v.at[buf], rrows_v.at[buf], trows_v.at[buf]
        for g in range(NG):

            def sbody(j, svec, g=g, hb=hb, rb=rb, tb=tb):
                s = g * L + j
                acc = jnp.zeros((L,), jnp.float32)
                for k in range(D // L):
                    sl = pl.ds(k * L, L)
                    acc = acc + jnp.abs(hb[s, sl] + rb[s, sl] - tb[s, sl])
                total = GAMMA - jnp.sum(acc)
                return jnp.where(iota == j, total, svec)

            svec = lax.fori_loop(0, L, sbody, jnp.zeros((L,), jnp.float32),
                                 unroll=2)
            score_v[pl.ds(g * L, L)] = svec
        pltpu.sync_copy(score_v, out_hbm.at[pl.ds(cb, CHUNK)])

    copies = stage(0, 0)
    for c in range(NCHUNK):
        nxt = stage(c + 1, (c + 1) % 2) if c + 1 < NCHUNK else None
        for cp in copies:
            cp.wait()
        score_chunk(c, c % 2)
        copies = nxt


def kernel(sample, entity_embedding, relation_embedding):
    scores = _sc_score(sample.astype(jnp.int32), entity_embedding,
                       relation_embedding)
    return scores[:, None]
